# baseline (speedup 1.0000x reference)
import functools
import jax
import jax.numpy as jnp
from jax import lax
from jax.experimental import pallas as pl
from jax.experimental.pallas import tpu as pltpu
from jax.experimental.pallas import tpu_sc as plsc

_D = 4096
_E = 64
_K = 8
_BT = 1024

_NC = 2
_NS = 16
_NW = _NC * _NS
_NSLICE = 4
_TPW = (32768 // _NSLICE) // _NW
_C = 256
_NCH = _TPW // _C


def _logits_body(x_ref, w_ref, out_ref):
    out_ref[...] = jnp.dot(x_ref[...], w_ref[...],
                           preferred_element_type=jnp.float32)


def _tc_logits(x, w):
    t = x.shape[0]
    return pl.pallas_call(
        _logits_body,
        grid=(t // _BT,),
        in_specs=[
            pl.BlockSpec((_BT, _D), lambda i: (i, 0)),
            pl.BlockSpec((_D, _E), lambda i: (0, 0)),
        ],
        out_specs=pl.BlockSpec((_BT, _E), lambda i: (i, 0)),
        out_shape=jax.ShapeDtypeStruct((t, _E), jnp.float32),
    )(x, w)


def _merge(a, b):
    av, ai = a
    bv, bi = b
    l8 = lax.iota(jnp.int32, 16) < 8
    mv = jnp.where(l8, av, lax.rev(bv, (0,)))
    mi = jnp.where(l8, ai, lax.rev(bi, (0,)))
    return plsc.sort_key_val(mv, mi, descending=True)


def _sc_router_body(logits_hbm, gates_hbm, idx_hbm, logits_v, dense_v, idx_v):
    wid = lax.axis_index("s") * _NC + lax.axis_index("c")
    iota16 = lax.iota(jnp.int32, 16)
    l8 = iota16 < 8

    for ch in range(_NCH):
        base = wid * _TPW + ch * _C
        pltpu.sync_copy(logits_hbm.at[pl.ds(base, _C), :], logits_v)

        def token_step(tloc, carry):
            parts = []
            for c in range(4):
                kv = logits_v[tloc, pl.ds(16 * c, 16)]
                parts.append(
                    plsc.sort_key_val(kv, iota16 + 16 * c, descending=True))
            sf_v, sf_i = _merge(_merge(parts[0], parts[1]),
                                _merge(parts[2], parts[3]))
            m = jnp.max(sf_v)
            e = jnp.exp(sf_v - m)
            z = jnp.sum(jnp.where(l8, e, 0.0))
            g = e / z
            row = jnp.full((16,), tloc, jnp.int32)
            for c in range(4):
                dense_v[tloc, pl.ds(16 * c, 16)] = jnp.zeros((16,), jnp.float32)
            plsc.store_scatter(dense_v, [row, sf_i], g, mask=l8)
            plsc.store_scatter(idx_v, [row, iota16], sf_i, mask=l8)
            return carry

        lax.fori_loop(0, _C, token_step, 0)

        pltpu.sync_copy(dense_v, gates_hbm.at[pl.ds(base, _C), :])
        pltpu.sync_copy(idx_v, idx_hbm.at[pl.ds(base, _C), :])


@jax.jit
def kernel(x, W_gate):
    t = x.shape[0]
    ts = t // _NSLICE
    sc_router = pl.kernel(
        _sc_router_body,
        out_type=[
            jax.ShapeDtypeStruct((ts, _E), jnp.float32),
            jax.ShapeDtypeStruct((ts, _K), jnp.int32),
        ],
        mesh=plsc.VectorSubcoreMesh(core_axis_name="c", subcore_axis_name="s"),
        scratch_types=[
            pltpu.VMEM((_C, _E), jnp.float32),
            pltpu.VMEM((_C, _E), jnp.float32),
            pltpu.VMEM((_C, _K), jnp.int32),
        ],
        compiler_params=pltpu.CompilerParams(needs_layout_passes=False),
    )
    parts = []
    for i in range(_NSLICE):
        logits_i = _tc_logits(lax.slice_in_dim(x, i * ts, (i + 1) * ts), W_gate)
        parts.append(sc_router(logits_i))
    gates = jnp.concatenate([p[0] for p in parts], axis=0)
    idx = jnp.concatenate([p[1] for p in parts], axis=0)
    return gates, idx


# speedup vs baseline: 3.0525x; 3.0525x over previous
import jax
import jax.numpy as jnp
from jax.experimental import pallas as pl

_D = 4096
_E = 64
_K = 8
_BT = 1024


def _router_body(x_ref, w_ref, gates_ref, idx_ref):
    x = x_ref[...]
    w = w_ref[...]
    logits = jnp.dot(x, w, preferred_element_type=jnp.float32)
    iota = jax.lax.broadcasted_iota(jnp.int32, logits.shape, 1)

    work = logits
    top_idx = []
    for _ in range(_K):
        idx = jnp.argmax(work, axis=-1)[:, None].astype(jnp.int32)
        top_idx.append(idx)
        work = jnp.where(iota == idx, -jnp.inf, work)

    v0 = jnp.max(logits, axis=-1, keepdims=True)
    e = jnp.where(work == -jnp.inf, jnp.exp(logits - v0), 0.0)
    gates_ref[...] = e / jnp.sum(e, axis=-1, keepdims=True)
    idx_ref[...] = jnp.concatenate(top_idx, axis=-1)


@jax.jit
def kernel(x, W_gate):
    t = x.shape[0]
    return pl.pallas_call(
        _router_body,
        grid=(t // _BT,),
        in_specs=[
            pl.BlockSpec((_BT, _D), lambda i: (i, 0)),
            pl.BlockSpec((_D, _E), lambda i: (0, 0)),
        ],
        out_specs=[
            pl.BlockSpec((_BT, _E), lambda i: (i, 0)),
            pl.BlockSpec((_BT, _K), lambda i: (i, 0)),
        ],
        out_shape=[
            jax.ShapeDtypeStruct((t, _E), jnp.float32),
            jax.ShapeDtypeStruct((t, _K), jnp.int32),
        ],
    )(x, W_gate)
